# Initial kernel scaffold; baseline (speedup 1.0000x reference)
import jax
import jax.numpy as jnp
from jax.experimental import pallas as pl


def kernel(emb_table, edge_index, edge_weight):
    raise NotImplementedError("write your pallas kernel here")


# speedup vs baseline: 13.1093x; 13.1093x over previous
import jax
import jax.numpy as jnp
from jax import lax
from jax.experimental import pallas as pl
from jax.experimental.pallas import tpu as pltpu
from jax.experimental.pallas import tpu_sc as plsc

N_NODES = 100001
D = 32
E = 1600000

NC = 2
NS = 16
L = 16

R_HALF = 51200
N_PAD = NC * R_HALF
ACC_ROWS = R_HALF + NS
WB_ROWS = R_HALF // NS
WB_BLK = 64
N_WB = WB_ROWS // WB_BLK

CH = 128
GR = 4
ET = 784
NG = ET // GR
ET_ALLOC = ET + 2 * GR
E_ROWS = NS * ET
DUMMY_DST = 2_000_000
WCH = GR * CH


def _make_layer(final: bool):
    mesh = plsc.VectorSubcoreMesh(
        core_axis_name="c", subcore_axis_name="s", num_cores=NC,
        num_subcores=NS)
    scratch = [
        pltpu.VMEM_SHARED((ACC_ROWS, D), jnp.float32),
        pltpu.VMEM((2 * GR, CH), jnp.int32),
        pltpu.VMEM((2 * GR, CH), jnp.int32),
        pltpu.VMEM((GR, CH), jnp.int32),
        pltpu.VMEM((2 * WCH + L,), jnp.float32),
        pltpu.VMEM((GR, CH, D), jnp.float32),
        pltpu.VMEM((WB_BLK, D), jnp.float32),
        pltpu.VMEM((WB_BLK, D), jnp.float32),
        pltpu.VMEM((WB_BLK, D), jnp.float32),
        pltpu.SemaphoreType.DMA((GR,)),
        pltpu.SemaphoreType.DMA((GR,)),
        pltpu.SemaphoreType.DMA,
    ]

    def body(*refs):
        if final:
            (emb_hbm, src_hbm, dst_hbm, w_hbm, zeros_hbm, emb0_hbm, out_hbm,
             acc_sh, src_v, dst_v, idx_v, w_v, rows_v, wb_v, wb0_v, wb1_v,
             sem_g, sem_s, sem_l) = refs
        else:
            (emb_hbm, src_hbm, dst_hbm, w_hbm, zeros_hbm, out_hbm,
             acc_sh, src_v, dst_v, idx_v, w_v, rows_v, wb_v, wb0_v, wb1_v,
             sem_g, sem_s, sem_l) = refs
        c = lax.axis_index("c")
        s = lax.axis_index("s")
        lo = c * R_HALF
        tile_base = s * ET_ALLOC

        pltpu.sync_copy(zeros_hbm, acc_sh.at[pl.ds(s * WB_ROWS, WB_ROWS)])
        plsc.subcore_barrier()

        lo16 = jnp.full((L,), lo, jnp.int32)
        hi16 = jnp.full((L,), lo + R_HALF, jnp.int32)
        dummy16 = jnp.full((L,), R_HALF + s, jnp.int32)

        def issue_idx(g, st):
            base = tile_base + g * GR
            pltpu.async_copy(src_hbm.at[pl.ds(base, GR)],
                             src_v.at[pl.ds(st * GR, GR)], sem_l)
            pltpu.async_copy(dst_hbm.at[pl.ds(base, GR)],
                             dst_v.at[pl.ds(st * GR, GR)], sem_l)
            pltpu.async_copy(w_hbm.at[pl.ds(base * CH, WCH)],
                             w_v.at[pl.ds(st * WCH, WCH)], sem_l)

        def wait_idx():
            pltpu.make_async_copy(
                src_hbm.at[pl.ds(0, GR)], src_v.at[pl.ds(0, GR)],
                sem_l).wait()
            pltpu.make_async_copy(
                dst_hbm.at[pl.ds(0, GR)], dst_v.at[pl.ds(0, GR)],
                sem_l).wait()
            pltpu.make_async_copy(
                w_hbm.at[pl.ds(0, WCH)], w_v.at[pl.ds(0, WCH)],
                sem_l).wait()

        def issue_gth(g, rowoff, st, b):
            del g
            pltpu.async_copy(emb_hbm.at[src_v.at[st * GR + rowoff]],
                             rows_v.at[b], sem_g.at[b])

        def wait_gth(b):
            pltpu.make_async_copy(
                emb_hbm.at[pl.ds(0, CH)], rows_v.at[b], sem_g.at[b]).wait()

        def issue_sct(b):
            pltpu.async_copy(rows_v.at[b], acc_sh.at[idx_v.at[b]],
                             sem_s.at[b], add=True)

        def wait_sct(b):
            pltpu.make_async_copy(
                emb_hbm.at[pl.ds(0, CH)], rows_v.at[b], sem_s.at[b]).wait()

        def remap(st, j):
            for k in range(CH // L):
                d16 = dst_v[st * GR + j, pl.ds(k * L, L)]
                inb = (d16 >= lo16) & (d16 < hi16)
                idx_v[j, pl.ds(k * L, L)] = jnp.where(inb, d16 - lo16,
                                                      dummy16)

        def scale(st, j):
            woff = st * WCH + j * CH

            def scale_k(k, _):
                w16 = w_v[pl.ds(woff + k * L, L)]
                for i in range(L):
                    ws = jnp.full((L,), w16[i])
                    e = k * L + i
                    x0 = rows_v[j, e, pl.ds(0, L)]
                    rows_v[j, e, pl.ds(0, L)] = x0 * ws
                    x1 = rows_v[j, e, pl.ds(L, L)]
                    rows_v[j, e, pl.ds(L, L)] = x1 * ws
                return 0
            lax.fori_loop(0, CH // L, scale_k, 0)

        def step(g, st, nst, j, first_groups: bool):
            nb = (j + 2) % GR
            if not (first_groups and j < 2):
                wait_sct(nb)
            if j == 2:
                wait_idx()
            if j < 2:
                issue_gth(g, j + 2, st, nb)
            else:
                issue_gth(g + 1, j - 2, nst, nb)
            wait_gth(j)
            remap(st, j)
            scale(st, j)
            issue_sct(j)

        issue_idx(0, 0)
        wait_idx()
        issue_gth(0, 0, 0, 0)
        issue_gth(0, 1, 0, 1)
        issue_idx(1, 1)
        for j in range(GR):
            step(0, 0, 1, j, first_groups=True)
        issue_idx(2, 0)
        for j in range(GR):
            step(1, 1, 0, j, first_groups=False)
        issue_idx(3, 1)

        def group_pair(gg, carry):
            g = 2 * gg
            for j in range(GR):
                step(g, 0, 1, j, first_groups=False)
            issue_idx(g + 2, 0)
            for j in range(GR):
                step(g + 1, 1, 0, j, first_groups=False)
            issue_idx(g + 3, 1)
            return carry
        lax.fori_loop(1, NG // 2, group_pair, 0)

        wait_gth(0)
        wait_gth(1)
        wait_sct(2)
        wait_sct(3)
        wait_idx()
        plsc.subcore_barrier()

        def wb_block(b, carry):
            row0 = s * WB_ROWS + b * WB_BLK
            if final:
                pltpu.sync_copy(acc_sh.at[pl.ds(row0, WB_BLK)], wb_v)
                pltpu.sync_copy(emb0_hbm.at[pl.ds(lo + row0, WB_BLK)], wb0_v)
                pltpu.sync_copy(emb_hbm.at[pl.ds(lo + row0, WB_BLK)], wb1_v)

                def mean_body(i, _):
                    for h in (0, L):
                        a2 = wb_v[i, pl.ds(h, L)]
                        a0 = wb0_v[i, pl.ds(h, L)]
                        a1 = wb1_v[i, pl.ds(h, L)]
                        wb_v[i, pl.ds(h, L)] = (a2 + a0 + a1) * jnp.float32(
                            1.0 / 3.0)
                    return 0
                lax.fori_loop(0, WB_BLK, mean_body, 0)
                pltpu.sync_copy(wb_v, out_hbm.at[pl.ds(lo + row0, WB_BLK)])
            else:
                pltpu.sync_copy(acc_sh.at[pl.ds(row0, WB_BLK)],
                                out_hbm.at[pl.ds(lo + row0, WB_BLK)])
            return carry
        lax.fori_loop(0, N_WB, wb_block, 0)

    return pl.kernel(
        body,
        out_type=jax.ShapeDtypeStruct((N_PAD, D), jnp.float32),
        mesh=mesh,
        scratch_types=scratch,
        compiler_params=pltpu.CompilerParams(use_tc_tiling_on_sc=False),
    )


_layer1 = _make_layer(final=False)
_layer2 = _make_layer(final=True)


def _pad_edges(x, fill):
    pad = NS * ET * CH - E
    x = jnp.concatenate([x, jnp.full((pad,), fill, x.dtype)])
    x = x.reshape(NS, ET, CH)
    x = jnp.pad(x, ((0, 0), (0, ET_ALLOC - ET), (0, 0)),
                constant_values=fill)
    return x.reshape(NS * ET_ALLOC, CH)


def kernel(emb_table, edge_index, edge_weight):
    src = edge_index[0].astype(jnp.int32)
    dst = edge_index[1].astype(jnp.int32)
    w = edge_weight.astype(jnp.float32)
    src_p = _pad_edges(src, 0)
    dst_p = _pad_edges(dst, DUMMY_DST)
    w_p = _pad_edges(w, 0.0).reshape(-1)
    emb0 = jnp.pad(emb_table, ((0, N_PAD - N_NODES), (0, 0)))
    zeros = jnp.zeros((WB_ROWS, D), jnp.float32)
    emb1 = _layer1(emb0, src_p, dst_p, w_p, zeros)
    out = _layer2(emb1, src_p, dst_p, w_p, zeros, emb0)
    return out[:N_NODES]


# speedup vs baseline: 13.1840x; 1.0057x over previous
import jax
import jax.numpy as jnp
from jax import lax
from jax.experimental import pallas as pl
from jax.experimental.pallas import tpu as pltpu
from jax.experimental.pallas import tpu_sc as plsc

N_NODES = 100001
D = 32
E = 1600000

NC = 2
NS = 16
L = 16

R_HALF = 51200
N_PAD = NC * R_HALF
NDUM = 16
ACC_ROWS = R_HALF + NS * NDUM
WB_ROWS = R_HALF // NS
WB_BLK = 64
N_WB = WB_ROWS // WB_BLK

CH = 128
GR = 4
ET = 784
NG = ET // GR
ET_ALLOC = ET + 2 * GR
DUMMY_DST = 2_000_000
WCH = GR * CH


def _make_layer(final: bool):
    mesh = plsc.VectorSubcoreMesh(
        core_axis_name="c", subcore_axis_name="s", num_cores=NC,
        num_subcores=NS)
    scratch = [
        pltpu.VMEM_SHARED((ACC_ROWS, D), jnp.float32),
        pltpu.VMEM((2 * GR, CH), jnp.int32),
        pltpu.VMEM((2 * GR, CH), jnp.int32),
        pltpu.VMEM((GR, CH), jnp.int32),
        pltpu.VMEM((2 * WCH + L,), jnp.float32),
        pltpu.VMEM((GR, CH, D), jnp.float32),
        pltpu.VMEM((WB_BLK, D), jnp.float32),
        pltpu.VMEM((WB_BLK, D), jnp.float32),
        pltpu.VMEM((WB_BLK, D), jnp.float32),
        pltpu.SemaphoreType.DMA((GR,)),
        pltpu.SemaphoreType.DMA((GR,)),
        pltpu.SemaphoreType.DMA,
    ]

    def body(*refs):
        if final:
            (emb_hbm, src_hbm, dst_hbm, w_hbm, zeros_hbm, emb0_hbm, out_hbm,
             acc_sh, src_v, dst_v, idx_v, w_v, rows_v, wb_v, wb0_v, wb1_v,
             sem_g, sem_s, sem_l) = refs
        else:
            (emb_hbm, src_hbm, dst_hbm, w_hbm, zeros_hbm, out_hbm,
             acc_sh, src_v, dst_v, idx_v, w_v, rows_v, wb_v, wb0_v, wb1_v,
             sem_g, sem_s, sem_l) = refs
        c = lax.axis_index("c")
        s = lax.axis_index("s")
        lo = c * R_HALF
        tile_base = s * ET_ALLOC

        pltpu.sync_copy(zeros_hbm, acc_sh.at[pl.ds(s * WB_ROWS, WB_ROWS)])
        plsc.subcore_barrier()

        lo16 = jnp.full((L,), lo, jnp.int32)
        hi16 = jnp.full((L,), lo + R_HALF, jnp.int32)
        dummy16 = jnp.full((L,), R_HALF + s * NDUM, jnp.int32) + (
            lax.iota(jnp.int32, L) & jnp.int32(NDUM - 1))

        def issue_idx(g, st):
            base = tile_base + g * GR
            pltpu.async_copy(src_hbm.at[pl.ds(base, GR)],
                             src_v.at[pl.ds(st * GR, GR)], sem_l)
            pltpu.async_copy(dst_hbm.at[pl.ds(base, GR)],
                             dst_v.at[pl.ds(st * GR, GR)], sem_l)
            pltpu.async_copy(w_hbm.at[pl.ds(base * CH, WCH)],
                             w_v.at[pl.ds(st * WCH, WCH)], sem_l)

        def wait_idx():
            pltpu.make_async_copy(
                src_hbm.at[pl.ds(0, GR)], src_v.at[pl.ds(0, GR)],
                sem_l).wait()
            pltpu.make_async_copy(
                dst_hbm.at[pl.ds(0, GR)], dst_v.at[pl.ds(0, GR)],
                sem_l).wait()
            pltpu.make_async_copy(
                w_hbm.at[pl.ds(0, WCH)], w_v.at[pl.ds(0, WCH)],
                sem_l).wait()

        def issue_gth(g, rowoff, st, b):
            del g
            pltpu.async_copy(emb_hbm.at[src_v.at[st * GR + rowoff]],
                             rows_v.at[b], sem_g.at[b])

        def wait_gth(b):
            pltpu.make_async_copy(
                emb_hbm.at[pl.ds(0, CH)], rows_v.at[b], sem_g.at[b]).wait()

        def issue_sct(b):
            pltpu.async_copy(rows_v.at[b], acc_sh.at[idx_v.at[b]],
                             sem_s.at[b], add=True)

        def wait_sct(b):
            pltpu.make_async_copy(
                emb_hbm.at[pl.ds(0, CH)], rows_v.at[b], sem_s.at[b]).wait()

        def remap(st, j):
            for k in range(CH // L):
                d16 = dst_v[st * GR + j, pl.ds(k * L, L)]
                inb = (d16 >= lo16) & (d16 < hi16)
                idx_v[j, pl.ds(k * L, L)] = jnp.where(inb, d16 - lo16,
                                                      dummy16)

        def scale(st, j):
            woff = st * WCH + j * CH

            def scale_k(k, _):
                w16 = w_v[pl.ds(woff + k * L, L)]
                for i in range(L):
                    ws = jnp.full((L,), w16[i])
                    e = k * L + i
                    x0 = rows_v[j, e, pl.ds(0, L)]
                    rows_v[j, e, pl.ds(0, L)] = x0 * ws
                    x1 = rows_v[j, e, pl.ds(L, L)]
                    rows_v[j, e, pl.ds(L, L)] = x1 * ws
                return 0
            lax.fori_loop(0, CH // L, scale_k, 0)

        def step(g, st, nst, j, first_groups: bool):
            nb = (j + 2) % GR
            if not (first_groups and j < 2):
                wait_sct(nb)
            if j == 2:
                wait_idx()
            if j < 2:
                issue_gth(g, j + 2, st, nb)
            else:
                issue_gth(g + 1, j - 2, nst, nb)
            wait_gth(j)
            remap(st, j)
            scale(st, j)
            issue_sct(j)

        issue_idx(0, 0)
        wait_idx()
        issue_gth(0, 0, 0, 0)
        issue_gth(0, 1, 0, 1)
        issue_idx(1, 1)
        for j in range(GR):
            step(0, 0, 1, j, first_groups=True)
        issue_idx(2, 0)
        for j in range(GR):
            step(1, 1, 0, j, first_groups=False)
        issue_idx(3, 1)

        def group_pair(gg, carry):
            g = 2 * gg
            for j in range(GR):
                step(g, 0, 1, j, first_groups=False)
            issue_idx(g + 2, 0)
            for j in range(GR):
                step(g + 1, 1, 0, j, first_groups=False)
            issue_idx(g + 3, 1)
            return carry
        lax.fori_loop(1, NG // 2, group_pair, 0)

        wait_gth(0)
        wait_gth(1)
        wait_sct(2)
        wait_sct(3)
        wait_idx()
        plsc.subcore_barrier()

        def wb_block(b, carry):
            row0 = s * WB_ROWS + b * WB_BLK
            if final:
                pltpu.sync_copy(acc_sh.at[pl.ds(row0, WB_BLK)], wb_v)
                pltpu.sync_copy(emb0_hbm.at[pl.ds(lo + row0, WB_BLK)], wb0_v)
                pltpu.sync_copy(emb_hbm.at[pl.ds(lo + row0, WB_BLK)], wb1_v)

                def mean_body(i, _):
                    for h in (0, L):
                        a2 = wb_v[i, pl.ds(h, L)]
                        a0 = wb0_v[i, pl.ds(h, L)]
                        a1 = wb1_v[i, pl.ds(h, L)]
                        wb_v[i, pl.ds(h, L)] = (a2 + a0 + a1) * jnp.float32(
                            1.0 / 3.0)
                    return 0
                lax.fori_loop(0, WB_BLK, mean_body, 0)
                pltpu.sync_copy(wb_v, out_hbm.at[pl.ds(lo + row0, WB_BLK)])
            else:
                pltpu.sync_copy(acc_sh.at[pl.ds(row0, WB_BLK)],
                                out_hbm.at[pl.ds(lo + row0, WB_BLK)])
            return carry
        lax.fori_loop(0, N_WB, wb_block, 0)

    return pl.kernel(
        body,
        out_type=jax.ShapeDtypeStruct((N_PAD, D), jnp.float32),
        mesh=mesh,
        scratch_types=scratch,
        compiler_params=pltpu.CompilerParams(use_tc_tiling_on_sc=False),
    )


_layer1 = _make_layer(final=False)
_layer2 = _make_layer(final=True)


def _pad_edges(x, fill):
    pad = NS * ET * CH - E
    x = jnp.concatenate([x, jnp.full((pad,), fill, x.dtype)])
    x = x.reshape(NS, ET, CH)
    x = jnp.pad(x, ((0, 0), (0, ET_ALLOC - ET), (0, 0)),
                constant_values=fill)
    return x.reshape(NS * ET_ALLOC, CH)


def kernel(emb_table, edge_index, edge_weight):
    src = edge_index[0].astype(jnp.int32)
    dst = edge_index[1].astype(jnp.int32)
    w = edge_weight.astype(jnp.float32)
    src_p = _pad_edges(src, 0)
    dst_p = _pad_edges(dst, DUMMY_DST)
    w_p = _pad_edges(w, 0.0).reshape(-1)
    emb0 = jnp.pad(emb_table, ((0, N_PAD - N_NODES), (0, 0)))
    zeros = jnp.zeros((WB_ROWS, D), jnp.float32)
    emb1 = _layer1(emb0, src_p, dst_p, w_p, zeros)
    out = _layer2(emb1, src_p, dst_p, w_p, zeros, emb0)
    return out[:N_NODES]
